# baseline (speedup 1.0000x reference)
import functools

import jax
import jax.numpy as jnp
from jax import lax
from jax.experimental import pallas as pl
from jax.experimental.pallas import tpu as pltpu
from jax.experimental.pallas import tpu_sc as plsc

_NUM_COLLISIONS = 1024
_SHIFT = 10
_MASK = _NUM_COLLISIONS - 1
_EMBED_DIM = 64
_ROW = 128
_BATCH = 16384
_NC = 2
_NS = 16
_NW = _NC * _NS
_BPW = _BATCH // _NW
_LANES = 16
_CHUNK = 128
_NCH = _BPW // _CHUNK


@functools.cache
def _build():
    @functools.partial(
        pl.kernel,
        out_type=jax.ShapeDtypeStruct((_EMBED_DIM, _BATCH), jnp.float32),
        mesh=plsc.VectorSubcoreMesh(core_axis_name="c", subcore_axis_name="s"),
        scratch_types=[
            pltpu.VMEM((_BPW,), jnp.int32),
            pltpu.VMEM((_NCH, _CHUNK), jnp.int32),
            pltpu.VMEM((_NCH, _CHUNK), jnp.int32),
            pltpu.VMEM((2, _CHUNK, _ROW), jnp.float32),
            pltpu.VMEM((2, _CHUNK, _ROW), jnp.float32),
            pltpu.VMEM((2, _EMBED_DIM, _CHUNK), jnp.float32),
            pltpu.VMEM((_CHUNK // _LANES, _LANES), jnp.int32),
            pltpu.SemaphoreType.DMA,
            pltpu.SemaphoreType.DMA,
            pltpu.SemaphoreType.DMA,
            pltpu.SemaphoreType.DMA,
            pltpu.SemaphoreType.DMA,
            pltpu.SemaphoreType.DMA,
        ],
        compiler_params=pltpu.CompilerParams(needs_layout_passes=False),
    )
    def _qr_embed(idx_hbm, wq_hbm, wr_hbm, out_hbm,
                  idx_v, q_v, r_v, bq, br, bo, rowtab,
                  sgq0, sgq1, sgr0, sgr1, sst0, sst1):
        wid = lax.axis_index("s") * _NC + lax.axis_index("c")
        base = wid * _BPW
        pltpu.sync_copy(idx_hbm.at[pl.ds(base, _BPW)], idx_v)

        sem_gq = (sgq0, sgq1)
        sem_gr = (sgr0, sgr1)
        sem_st = (sst0, sst1)
        jota = lax.iota(jnp.int32, _LANES)
        for g in range(_CHUNK // _LANES):
            rowtab[g, :] = jota + g * _LANES

        def split(c):
            def body(i, carry):
                sl = pl.ds(i * _LANES, _LANES)
                v = idx_v[pl.ds(c * _CHUNK + i * _LANES, _LANES)]
                q_v[c, sl] = lax.shift_right_logical(v, _SHIFT)
                r_v[c, sl] = lax.bitwise_and(v, _MASK)
                return carry
            lax.fori_loop(0, _CHUNK // _LANES, body, 0)

        def start_gathers(c):
            b = c % 2
            cq = pltpu.async_copy(wq_hbm.at[q_v.at[c]], bq.at[b], sem_gq[b])
            cr = pltpu.async_copy(wr_hbm.at[r_v.at[c]], br.at[b], sem_gr[b])
            return cq, cr

        split(0)
        pending = {0: start_gathers(0)}
        stores = {}

        for c in range(_NCH):
            b = c % 2
            if c + 1 < _NCH:
                if c - 1 >= 0:
                    stores.pop(c - 1).wait()
                split(c + 1)
                pending[c + 1] = start_gathers(c + 1)
            cq, cr = pending.pop(c)
            cq.wait()
            cr.wait()

            def grp_body(g, carry):
                rowv = rowtab[g, :]
                sl = pl.ds(g * _LANES, _LANES)
                for j in range(_EMBED_DIM):
                    colv = jnp.full((_LANES,), j, jnp.int32)
                    cq = plsc.load_gather(bq.at[b], [rowv, colv])
                    cr = plsc.load_gather(br.at[b], [rowv, colv])
                    bo[b, j, sl] = cq * cr
                return carry

            lax.fori_loop(0, _CHUNK // _LANES, grp_body, 0)

            stores[c] = pltpu.async_copy(
                bo.at[b],
                out_hbm.at[:, pl.ds(base + c * _CHUNK, _CHUNK)],
                sem_st[b])

        for c in sorted(stores):
            stores.pop(c).wait()

    return _qr_embed


def kernel(input, weight_q, weight_r):
    wq = jnp.pad(weight_q, ((0, 0), (0, _ROW - _EMBED_DIM)))
    wr = jnp.pad(weight_r, ((0, 0), (0, _ROW - _EMBED_DIM)))
    return _build()(input, wq, wr).T


# speedup vs baseline: 1.2917x; 1.2917x over previous
import functools

import jax
import jax.numpy as jnp
from jax import lax
from jax.experimental import pallas as pl
from jax.experimental.pallas import tpu as pltpu
from jax.experimental.pallas import tpu_sc as plsc

_NUM_COLLISIONS = 1024
_SHIFT = 10
_MASK = _NUM_COLLISIONS - 1
_EMBED_DIM = 64
_ROW = 128
_BATCH = 16384
_NC = 2
_NS = 16
_NW = _NC * _NS
_BPW = _BATCH // _NW
_LANES = 16
_CHUNK = 128
_NCH = _BPW // _CHUNK


@functools.cache
def _build():
    @functools.partial(
        pl.kernel,
        out_type=jax.ShapeDtypeStruct((_EMBED_DIM, _BATCH), jnp.float32),
        mesh=plsc.VectorSubcoreMesh(core_axis_name="c", subcore_axis_name="s"),
        scratch_types=[
            pltpu.VMEM((_BPW,), jnp.int32),
            pltpu.VMEM((_NCH, _CHUNK), jnp.int32),
            pltpu.VMEM((_NCH, _CHUNK), jnp.int32),
            pltpu.VMEM((2, _CHUNK, _ROW), jnp.float32),
            pltpu.VMEM((2, _CHUNK, _ROW), jnp.float32),
            pltpu.VMEM((2, _EMBED_DIM, _CHUNK), jnp.float32),
            pltpu.VMEM((_CHUNK // _LANES, _LANES), jnp.int32),
            pltpu.VMEM((_LANES, _LANES), jnp.int32),
            pltpu.SemaphoreType.DMA,
            pltpu.SemaphoreType.DMA,
            pltpu.SemaphoreType.DMA,
            pltpu.SemaphoreType.DMA,
            pltpu.SemaphoreType.DMA,
            pltpu.SemaphoreType.DMA,
        ],
        compiler_params=pltpu.CompilerParams(needs_layout_passes=False),
    )
    def _qr_embed(idx_hbm, wq_hbm, wr_hbm, out_hbm,
                  idx_v, q_v, r_v, bq, br, bo, rowtab, rottab,
                  sgq0, sgq1, sgr0, sgr1, sst0, sst1):
        wid = lax.axis_index("s") * _NC + lax.axis_index("c")
        base = wid * _BPW
        pltpu.sync_copy(idx_hbm.at[pl.ds(base, _BPW)], idx_v)

        sem_gq = (sgq0, sgq1)
        sem_gr = (sgr0, sgr1)
        sem_st = (sst0, sst1)
        jota = lax.iota(jnp.int32, _LANES)
        for g in range(_CHUNK // _LANES):
            rowtab[g, :] = jota + g * _LANES
        for d in range(_LANES):
            rottab[d, :] = lax.bitwise_and(jota + d, _LANES - 1)

        def split(c):
            def body(i, carry):
                sl = pl.ds(i * _LANES, _LANES)
                v = idx_v[pl.ds(c * _CHUNK + i * _LANES, _LANES)]
                q_v[c, sl] = lax.shift_right_logical(v, _SHIFT)
                r_v[c, sl] = lax.bitwise_and(v, _MASK)
                return carry
            lax.fori_loop(0, _CHUNK // _LANES, body, 0)

        def start_gathers(c):
            b = c % 2
            cq = pltpu.async_copy(wq_hbm.at[q_v.at[c]], bq.at[b], sem_gq[b])
            cr = pltpu.async_copy(wr_hbm.at[r_v.at[c]], br.at[b], sem_gr[b])
            return cq, cr

        split(0)
        pending = {0: start_gathers(0)}
        stores = {}

        for c in range(_NCH):
            b = c % 2
            if c + 1 < _NCH:
                if c - 1 >= 0:
                    stores.pop(c - 1).wait()
                split(c + 1)
                pending[c + 1] = start_gathers(c + 1)
            cq, cr = pending.pop(c)
            cq.wait()
            cr.wait()

            def grp_body(g, carry):
                rowv = rowtab[g, :]
                for jg in range(_EMBED_DIM // _LANES):
                    for d in range(_LANES):
                        colv = rottab[d, :] + jg * _LANES
                        cq = plsc.load_gather(bq.at[b], [rowv, colv])
                        cr = plsc.load_gather(br.at[b], [rowv, colv])
                        plsc.store_scatter(bo.at[b], [colv, rowv], cq * cr)
                return carry

            lax.fori_loop(0, _CHUNK // _LANES, grp_body, 0)

            stores[c] = pltpu.async_copy(
                bo.at[b],
                out_hbm.at[:, pl.ds(base + c * _CHUNK, _CHUNK)],
                sem_st[b])

        for c in sorted(stores):
            stores.pop(c).wait()

    return _qr_embed


def kernel(input, weight_q, weight_r):
    wq = jnp.pad(weight_q, ((0, 0), (0, _ROW - _EMBED_DIM)))
    wr = jnp.pad(weight_r, ((0, 0), (0, _ROW - _EMBED_DIM)))
    return _build()(input, wq, wr).T


# speedup vs baseline: 1.7501x; 1.3548x over previous
import functools

import jax
import jax.numpy as jnp
from jax import lax
from jax.experimental import pallas as pl
from jax.experimental.pallas import tpu as pltpu
from jax.experimental.pallas import tpu_sc as plsc

_NUM_COLLISIONS = 1024
_SHIFT = 10
_MASK = _NUM_COLLISIONS - 1
_EMBED_DIM = 64
_ROW = 128
_BATCH = 16384
_NC = 2
_NS = 16
_NW = _NC * _NS
_BPW = _BATCH // _NW
_LANES = 16
_CHUNK = 128
_NCH = _BPW // _CHUNK


@functools.cache
def _build():
    @functools.partial(
        pl.kernel,
        out_type=jax.ShapeDtypeStruct((_BATCH, _EMBED_DIM), jnp.float32),
        mesh=plsc.VectorSubcoreMesh(core_axis_name="c", subcore_axis_name="s"),
        scratch_types=[
            pltpu.VMEM((_BPW,), jnp.int32),
            pltpu.VMEM((_NCH, _CHUNK), jnp.int32),
            pltpu.VMEM((_NCH, _CHUNK), jnp.int32),
            pltpu.VMEM((2, _CHUNK, _ROW), jnp.float32),
            pltpu.VMEM((2, _CHUNK, _ROW), jnp.float32),
            pltpu.VMEM((2, _CHUNK, _EMBED_DIM), jnp.float32),
            pltpu.SemaphoreType.DMA,
            pltpu.SemaphoreType.DMA,
            pltpu.SemaphoreType.DMA,
            pltpu.SemaphoreType.DMA,
            pltpu.SemaphoreType.DMA,
            pltpu.SemaphoreType.DMA,
        ],
    )
    def _qr_embed(idx_hbm, wq_hbm, wr_hbm, out_hbm,
                  idx_v, q_v, r_v, bq, br, bo,
                  sgq0, sgq1, sgr0, sgr1, sst0, sst1):
        wid = lax.axis_index("s") * _NC + lax.axis_index("c")
        base = wid * _BPW
        pltpu.sync_copy(idx_hbm.at[pl.ds(base, _BPW)], idx_v)

        sem_gq = (sgq0, sgq1)
        sem_gr = (sgr0, sgr1)
        sem_st = (sst0, sst1)

        def split(c):
            def body(i, carry):
                sl = pl.ds(i * _LANES, _LANES)
                v = idx_v[pl.ds(c * _CHUNK + i * _LANES, _LANES)]
                q_v[c, sl] = lax.shift_right_logical(v, _SHIFT)
                r_v[c, sl] = lax.bitwise_and(v, _MASK)
                return carry
            lax.fori_loop(0, _CHUNK // _LANES, body, 0)

        def start_gathers(c):
            b = c % 2
            cq = pltpu.async_copy(wq_hbm.at[q_v.at[c]], bq.at[b], sem_gq[b])
            cr = pltpu.async_copy(wr_hbm.at[r_v.at[c]], br.at[b], sem_gr[b])
            return cq, cr

        split(0)
        pending = {0: start_gathers(0)}
        stores = {}

        for c in range(_NCH):
            b = c % 2
            if c + 1 < _NCH:
                if c - 1 >= 0:
                    stores.pop(c - 1).wait()
                split(c + 1)
                pending[c + 1] = start_gathers(c + 1)
            cq, cr = pending.pop(c)
            cq.wait()
            cr.wait()

            def mul_body(row, carry):
                for j in range(_EMBED_DIM // _LANES):
                    sl = pl.ds(j * _LANES, _LANES)
                    bo[b, row, sl] = bq[b, row, sl] * br[b, row, sl]
                return carry

            lax.fori_loop(0, _CHUNK, mul_body, 0)

            stores[c] = pltpu.async_copy(
                bo.at[b], out_hbm.at[pl.ds(base + c * _CHUNK, _CHUNK)],
                sem_st[b])

        for c in sorted(stores):
            stores.pop(c).wait()

    return _qr_embed


def kernel(input, weight_q, weight_r):
    wq = jnp.pad(weight_q, ((0, 0), (0, _ROW - _EMBED_DIM)))
    wr = jnp.pad(weight_r, ((0, 0), (0, _ROW - _EMBED_DIM)))
    return _build()(input, wq, wr)
